# baseline (speedup 1.0000x reference)
import functools

import jax
import jax.numpy as jnp
from jax import lax
from jax.experimental import pallas as pl
from jax.experimental.pallas import tpu as pltpu
from jax.experimental.pallas import tpu_sc as plsc

BATCH = 16384
EMB_DIM = 32
LANES = 16
ROW_ELEMS = 16
NB_ROWS = 1000000
J_STRIDE = NB_ROWS // ROW_ELEMS

_info = plsc.get_sparse_core_info()
NUM_CORES = _info.num_cores
NUM_SUBCORES = _info.num_subcores
NUM_WORKERS = NUM_CORES * NUM_SUBCORES
B_PER_W = BATCH // NUM_WORKERS
CHUNK = 128
N_CHUNKS = B_PER_W // CHUNK

_mesh = plsc.VectorSubcoreMesh(core_axis_name="c", subcore_axis_name="s")


@functools.partial(
    pl.kernel,
    mesh=_mesh,
    compiler_params=pltpu.CompilerParams(use_tc_tiling_on_sc=False,
                                         needs_layout_passes=False),
    out_type=jax.ShapeDtypeStruct((EMB_DIM, BATCH), jnp.float32),
    scratch_types=[
        pltpu.VMEM((B_PER_W,), jnp.int32),
        pltpu.VMEM((B_PER_W,), jnp.int32),
        pltpu.VMEM((B_PER_W,), jnp.int32),
        pltpu.VMEM((B_PER_W,), jnp.int32),
        pltpu.VMEM((2, B_PER_W), jnp.int32),
        pltpu.VMEM((2, B_PER_W), jnp.int32),
        pltpu.VMEM((2, N_CHUNKS, CHUNK, ROW_ELEMS), jnp.float32),
        pltpu.VMEM((2, N_CHUNKS, CHUNK, ROW_ELEMS), jnp.float32),
        pltpu.VMEM((EMB_DIM, B_PER_W), jnp.float32),
        pltpu.VMEM((EMB_DIM, B_PER_W), jnp.float32),
        pltpu.SemaphoreType.DMA,
        pltpu.SemaphoreType.DMA,
    ],
)
def _gmf_sc(uidx_hbm, iidx_hbm, utab_hbm, itab_hbm, out_hbm,
            urow, irow, ulane, ilane, uidx, iidx, ug, ig, ublk, iblk,
            usem, isem):
  wid = lax.axis_index("s") * NUM_CORES + lax.axis_index("c")
  base = wid * B_PER_W

  pltpu.sync_copy(uidx_hbm.at[pl.ds(base, B_PER_W)], urow)
  pltpu.sync_copy(iidx_hbm.at[pl.ds(base, B_PER_W)], irow)

  def split_body(v, carry):
    sl = pl.ds(v * LANES, LANES)
    uv = urow[sl]
    iv = irow[sl]
    ulane[sl] = uv & (ROW_ELEMS - 1)
    ilane[sl] = iv & (ROW_ELEMS - 1)
    urow[sl] = uv >> 4
    irow[sl] = iv >> 4
    return carry

  lax.fori_loop(0, B_PER_W // LANES, split_body, 0, unroll=4)

  pos_v = lax.iota(jnp.int32, 16)

  def fire(j, s):
    joff = j * J_STRIDE

    def idx_body(v, carry):
      sl = pl.ds(v * LANES, LANES)
      uidx[s, sl] = urow[sl] + joff
      iidx[s, sl] = irow[sl] + joff
      return carry

    lax.fori_loop(0, B_PER_W // LANES, idx_body, 0, unroll=4)
    for k in range(N_CHUNKS):
      sl = pl.ds(k * CHUNK, CHUNK)
      pltpu.async_copy(utab_hbm.at[uidx.at[s, sl]], ug.at[s, k], usem)
      pltpu.async_copy(itab_hbm.at[iidx.at[s, sl]], ig.at[s, k], isem)

  def drain(s):
    for k in range(N_CHUNKS):
      pltpu.make_async_copy(utab_hbm.at[pl.ds(0, CHUNK)], ug.at[s, k],
                            usem).wait()
      pltpu.make_async_copy(itab_hbm.at[pl.ds(0, CHUNK)], ig.at[s, k],
                            isem).wait()

  def extract(j, s):
    for k in range(N_CHUNKS):
      for p in range(CHUNK // LANES):
        off = k * CHUNK + p * LANES
        sl = pl.ds(off, LANES)
        pv = pos_v + p * LANES
        ublk[j, sl] = plsc.load_gather(ug.at[s, k], [pv, ulane[sl]])
        iblk[j, sl] = plsc.load_gather(ig.at[s, k], [pv, ilane[sl]])

  fire(0, 0)

  def pipe_body(j, carry):
    s = j & 1
    drain(s)
    fire(j + 1, 1 - s)
    extract(j, s)
    return carry

  lax.fori_loop(0, EMB_DIM - 1, pipe_body, 0)
  s_last = (EMB_DIM - 1) & 1
  drain(s_last)
  extract(EMB_DIM - 1, s_last)

  def mul_body(t, carry):
    j = t >> 5
    sl = pl.ds((t & 31) * LANES, LANES)
    ublk[j, sl] = ublk[j, sl] * iblk[j, sl]
    return carry

  lax.fori_loop(0, (EMB_DIM * B_PER_W) // LANES, mul_body, 0, unroll=8)

  pltpu.sync_copy(ublk, out_hbm.at[:, pl.ds(base, B_PER_W)])


def kernel(user_idx, item_idx, emb_user, emb_item):
  return _gmf_sc(user_idx.astype(jnp.int32), item_idx.astype(jnp.int32),
                 emb_user.T.reshape(-1, ROW_ELEMS),
                 emb_item.T.reshape(-1, ROW_ELEMS)).T


# speedup vs baseline: 18.3932x; 18.3932x over previous
import functools

import jax
import jax.numpy as jnp
from jax import lax
from jax.experimental import pallas as pl
from jax.experimental.pallas import tpu as pltpu
from jax.experimental.pallas import tpu_sc as plsc

BATCH = 16384
EMB_DIM = 32
LANES = 16

_info = plsc.get_sparse_core_info()
NUM_CORES = _info.num_cores
NUM_SUBCORES = _info.num_subcores
NUM_WORKERS = NUM_CORES * NUM_SUBCORES
B_PER_W = BATCH // NUM_WORKERS
SUB = 4
PHASES = LANES // SUB
N_SUPER = B_PER_W // LANES

_mesh = plsc.VectorSubcoreMesh(core_axis_name="c", subcore_axis_name="s")


@functools.partial(
    pl.kernel,
    mesh=_mesh,
    compiler_params=pltpu.CompilerParams(use_tc_tiling_on_sc=True,
                                         needs_layout_passes=False),
    out_type=jax.ShapeDtypeStruct((EMB_DIM, BATCH), jnp.float32),
    scratch_types=[
        pltpu.VMEM((B_PER_W,), jnp.int32),
        pltpu.VMEM((B_PER_W,), jnp.int32),
        pltpu.VMEM((2, SUB, EMB_DIM, 128), jnp.float32),
        pltpu.VMEM((2, SUB, EMB_DIM, 128), jnp.float32),
        pltpu.VMEM((EMB_DIM, B_PER_W), jnp.float32),
        pltpu.SemaphoreType.DMA,
        pltpu.SemaphoreType.DMA,
    ],
)
def _gmf_sc(uidx_hbm, iidx_hbm, utab_hbm, itab_hbm, out_hbm,
            uixv, iixv, uwin, iwin, obuf, usem, isem):
  wid = lax.axis_index("s") * NUM_CORES + lax.axis_index("c")
  base = wid * B_PER_W

  pltpu.sync_copy(uidx_hbm.at[pl.ds(base, B_PER_W)], uixv)
  pltpu.sync_copy(iidx_hbm.at[pl.ds(base, B_PER_W)], iixv)

  j_lo = lax.iota(jnp.int32, 16)
  j_hi = j_lo + 16
  zeros = jnp.full((16,), 0, jnp.int32)

  def super_body(t, carry):
    uvec = uixv[pl.ds(t * LANES, LANES)]
    ivec = iixv[pl.ds(t * LANES, LANES)]
    u_sc = [uvec[q] for q in range(LANES)]
    i_sc = [ivec[q] for q in range(LANES)]

    def fire(phase):
      s = phase & 1
      for k in range(SUB):
        q = phase * SUB + k
        cu = pl.multiple_of((u_sc[q] >> 7) << 7, 128)
        ci = pl.multiple_of((i_sc[q] >> 7) << 7, 128)
        pltpu.async_copy(utab_hbm.at[:, pl.ds(cu, 128)], uwin.at[s, k], usem)
        pltpu.async_copy(itab_hbm.at[:, pl.ds(ci, 128)], iwin.at[s, k], isem)

    def drain(phase):
      s = phase & 1
      for k in range(SUB):
        pltpu.make_async_copy(utab_hbm.at[:, pl.ds(0, 128)], uwin.at[s, k],
                              usem).wait()
        pltpu.make_async_copy(itab_hbm.at[:, pl.ds(0, 128)], iwin.at[s, k],
                              isem).wait()

    def extract(phase):
      s = phase & 1
      for k in range(SUB):
        q = phase * SUB + k
        lu = zeros + (u_sc[q] & 127)
        li = zeros + (i_sc[q] & 127)
        bv = zeros + (t * LANES + q)
        u_lo = plsc.load_gather(uwin.at[s, k], [j_lo, lu])
        i_lo = plsc.load_gather(iwin.at[s, k], [j_lo, li])
        plsc.store_scatter(obuf, [j_lo, bv], u_lo * i_lo)
        u_hi = plsc.load_gather(uwin.at[s, k], [j_hi, lu])
        i_hi = plsc.load_gather(iwin.at[s, k], [j_hi, li])
        plsc.store_scatter(obuf, [j_hi, bv], u_hi * i_hi)

    fire(0)
    for phase in range(PHASES):
      if phase + 1 < PHASES:
        drain(phase)
        fire(phase + 1)
        extract(phase)
      else:
        drain(phase)
        extract(phase)
    return carry

  lax.fori_loop(0, N_SUPER, super_body, 0)

  pltpu.sync_copy(obuf, out_hbm.at[:, pl.ds(base, B_PER_W)])


def kernel(user_idx, item_idx, emb_user, emb_item):
  return _gmf_sc(user_idx.astype(jnp.int32), item_idx.astype(jnp.int32),
                 emb_user.T, emb_item.T).T
